# Initial kernel scaffold; baseline (speedup 1.0000x reference)
import jax
import jax.numpy as jnp
from jax.experimental import pallas as pl


def kernel(workloads, w1, b1, g1, be1, w2, b2, g2, be2):
    raise NotImplementedError("write your pallas kernel here")


# speedup vs baseline: 669.4043x; 669.4043x over previous
import jax
import jax.numpy as jnp
from jax.experimental import pallas as pl
from jax.experimental.pallas import tpu as pltpu

NBINS = 10
HID = 128
S_COLS = 16384
BR = 128
C = 512
G = 15
LN_EPS = 1e-5


def _accum_group(x_ref, g_base, nchunks, counts):
    acc = jnp.zeros((BR, C), jnp.int32)
    c9a = jnp.zeros((BR, C), jnp.float32)
    for j in range(nchunks):
        xc = x_ref[:, pl.ds(g_base + j * C, C)]
        q = xc.astype(jnp.int32)
        sh = q << 2
        p = jnp.where(xc < 8.0, jnp.int32(1) << sh, 0)
        acc = acc + p
        c9a = c9a + jnp.where(xc < 9.0, 1.0, 0.0)
    new = []
    for b in range(8):
        f = ((acc >> (4 * b)) & 15) if b else (acc & 15)
        new.append(counts[b] + jnp.sum(f.astype(jnp.float32), axis=-1, keepdims=True))
    new.append(counts[8] + jnp.sum(c9a, axis=-1, keepdims=True))
    return tuple(new)


def _body(x_ref, w1_ref, b1_ref, g1_ref, be1_ref, w2_ref, b2_ref, g2_ref,
          be2_ref, o_ref):
    S = x_ref.shape[1]
    nchunks = S // C
    ngroups = nchunks // G
    tail = nchunks - ngroups * G

    init = tuple(jnp.zeros((BR, 1), jnp.float32) for _ in range(9))
    counts = jax.lax.fori_loop(
        0, ngroups, lambda g, c: _accum_group(x_ref, g * (G * C), G, c), init)
    if tail:
        counts = _accum_group(x_ref, ngroups * G * C, tail, counts)

    inv = jnp.float32(1.0 / (S_COLS + 1e-8))
    low_sum = counts[0]
    for b in range(1, 8):
        low_sum = low_sum + counts[b]
    hist = list(counts[:8])
    hist.append(counts[8] - low_sum)
    hist.append(jnp.full((BR, 1), float(S_COLS), jnp.float32) - counts[8])

    a1 = jnp.zeros((BR, HID), jnp.float32)
    for b in range(NBINS):
        a1 = a1 + (hist[b] * inv) * w1_ref[b:b + 1, :]
    a1 = jnp.maximum(a1 + b1_ref[...], 0.0)
    m1 = jnp.mean(a1, axis=-1, keepdims=True)
    v1 = jnp.mean((a1 - m1) ** 2, axis=-1, keepdims=True)
    h1 = (a1 - m1) * jax.lax.rsqrt(v1 + LN_EPS) * g1_ref[...] + be1_ref[...]

    a2 = jnp.dot(h1, w2_ref[...], preferred_element_type=jnp.float32)
    a2 = jnp.maximum(a2 + b2_ref[...], 0.0)
    m2 = jnp.mean(a2, axis=-1, keepdims=True)
    v2 = jnp.mean((a2 - m2) ** 2, axis=-1, keepdims=True)
    o_ref[...] = (a2 - m2) * jax.lax.rsqrt(v2 + LN_EPS) * g2_ref[...] + be2_ref[...]


def kernel(workloads, w1, b1, g1, be1, w2, b2, g2, be2):
    B, S = workloads.shape
    grid = (B // BR,)
    vec = lambda v: v.reshape(1, HID)
    out = pl.pallas_call(
        _body,
        grid=grid,
        in_specs=[
            pl.BlockSpec((BR, S), lambda i: (i, 0)),
            pl.BlockSpec((NBINS, HID), lambda i: (0, 0)),
            pl.BlockSpec((1, HID), lambda i: (0, 0)),
            pl.BlockSpec((1, HID), lambda i: (0, 0)),
            pl.BlockSpec((1, HID), lambda i: (0, 0)),
            pl.BlockSpec((HID, HID), lambda i: (0, 0)),
            pl.BlockSpec((1, HID), lambda i: (0, 0)),
            pl.BlockSpec((1, HID), lambda i: (0, 0)),
            pl.BlockSpec((1, HID), lambda i: (0, 0)),
        ],
        out_specs=pl.BlockSpec((BR, HID), lambda i: (i, 0)),
        out_shape=jax.ShapeDtypeStruct((B, HID), jnp.float32),
        compiler_params=pltpu.CompilerParams(
            dimension_semantics=("parallel",),
            vmem_limit_bytes=100 * 1024 * 1024,
        ),
    )(workloads, w1, vec(b1), vec(g1), vec(be1), w2, vec(b2), vec(g2), vec(be2))
    return out


# speedup vs baseline: 697.3053x; 1.0417x over previous
import jax
import jax.numpy as jnp
from jax.experimental import pallas as pl
from jax.experimental.pallas import tpu as pltpu

NBINS = 10
HID = 128
S_COLS = 16384
BR = 128
C = 512
G = 7
LN_EPS = 1e-5


def _accum_group(xa_ref, xb_ref, g_base, nchunks, counts):
    acc = jnp.zeros((BR, C), jnp.int32)
    accq = jnp.zeros((BR, C), jnp.int32)
    for j in range(nchunks):
        for ref in (xa_ref, xb_ref):
            xc = ref[:, pl.ds(g_base + j * C, C)]
            q = xc.astype(jnp.int32)
            sh = q << 2
            p = jnp.where(xc < 8.0, jnp.int32(1) << sh, 0)
            acc = acc + p
            accq = accq + q
    new = []
    for b in range(8):
        f = ((acc >> (4 * b)) & 15) if b else (acc & 15)
        new.append(counts[b] + jnp.sum(f, axis=-1, keepdims=True))
    new.append(counts[8] + jnp.sum(accq, axis=-1, keepdims=True))
    return tuple(new)


def _body(xa_ref, xb_ref, w1_ref, b1_ref, g1_ref, be1_ref, w2_ref, b2_ref,
          g2_ref, be2_ref, o_ref):
    S = xa_ref.shape[1]
    nchunks = S // C
    ngroups = nchunks // G
    tail = nchunks - ngroups * G

    init = tuple(jnp.zeros((BR, 1), jnp.int32) for _ in range(9))
    counts = jax.lax.fori_loop(
        0, ngroups, lambda g, c: _accum_group(xa_ref, xb_ref, g * (G * C), G, c),
        init)
    if tail:
        counts = _accum_group(xa_ref, xb_ref, ngroups * G * C, tail, counts)

    inv = jnp.float32(1.0 / (S_COLS + 1e-8))
    lowf = [c.astype(jnp.float32) for c in counts[:8]]
    qsum = counts[8].astype(jnp.float32)
    low_sum = lowf[0]
    low_wsum = jnp.zeros((BR, 1), jnp.float32)
    for b in range(1, 8):
        low_sum = low_sum + lowf[b]
        low_wsum = low_wsum + jnp.float32(b) * lowf[b]
    n89 = jnp.full((BR, 1), float(S_COLS), jnp.float32) - low_sum
    c9 = qsum - low_wsum - 8.0 * n89
    hist = lowf
    hist.append(n89 - c9)
    hist.append(c9)

    a1 = jnp.zeros((BR, HID), jnp.float32)
    for b in range(NBINS):
        a1 = a1 + (hist[b] * inv) * w1_ref[b:b + 1, :]
    a1 = jnp.maximum(a1 + b1_ref[...], 0.0)
    m1 = jnp.mean(a1, axis=-1, keepdims=True)
    v1 = jnp.mean((a1 - m1) ** 2, axis=-1, keepdims=True)
    h1 = (a1 - m1) * jax.lax.rsqrt(v1 + LN_EPS) * g1_ref[...] + be1_ref[...]

    a2 = jnp.dot(h1, w2_ref[...], preferred_element_type=jnp.float32)
    a2 = jnp.maximum(a2 + b2_ref[...], 0.0)
    m2 = jnp.mean(a2, axis=-1, keepdims=True)
    v2 = jnp.mean((a2 - m2) ** 2, axis=-1, keepdims=True)
    o_ref[...] = (a2 - m2) * jax.lax.rsqrt(v2 + LN_EPS) * g2_ref[...] + be2_ref[...]


def kernel(workloads, w1, b1, g1, be1, w2, b2, g2, be2):
    B, S = workloads.shape
    grid = (B // BR,)
    half = S // 2
    vec = lambda v: v.reshape(1, HID)
    out = pl.pallas_call(
        _body,
        grid=grid,
        in_specs=[
            pl.BlockSpec((BR, half), lambda i: (i, 0)),
            pl.BlockSpec((BR, half), lambda i: (i, 1)),
            pl.BlockSpec((NBINS, HID), lambda i: (0, 0)),
            pl.BlockSpec((1, HID), lambda i: (0, 0)),
            pl.BlockSpec((1, HID), lambda i: (0, 0)),
            pl.BlockSpec((1, HID), lambda i: (0, 0)),
            pl.BlockSpec((HID, HID), lambda i: (0, 0)),
            pl.BlockSpec((1, HID), lambda i: (0, 0)),
            pl.BlockSpec((1, HID), lambda i: (0, 0)),
            pl.BlockSpec((1, HID), lambda i: (0, 0)),
        ],
        out_specs=pl.BlockSpec((BR, HID), lambda i: (i, 0)),
        out_shape=jax.ShapeDtypeStruct((B, HID), jnp.float32),
        compiler_params=pltpu.CompilerParams(
            dimension_semantics=("parallel",),
            vmem_limit_bytes=100 * 1024 * 1024,
        ),
    )(workloads, workloads, w1, vec(b1), vec(g1), vec(be1), w2, vec(b2),
      vec(g2), vec(be2))
    return out


# speedup vs baseline: 730.9788x; 1.0483x over previous
import jax
import jax.numpy as jnp
from jax.experimental import pallas as pl
from jax.experimental.pallas import tpu as pltpu

NBINS = 10
HID = 128
S_COLS = 16384
BR = 256
C = 512
G = 15
LN_EPS = 1e-5


def _accum_group(x_ref, g_base, nchunks, counts):
    acc = jnp.zeros((BR, C), jnp.int32)
    accq = jnp.zeros((BR, C), jnp.int32)
    for j in range(nchunks):
        xc = x_ref[:, pl.ds(g_base + j * C, C)]
        q = xc.astype(jnp.int32)
        sh = q << 2
        p = jnp.where(xc < 8.0, jnp.int32(1) << sh, 0)
        acc = acc + p
        accq = accq + q
    new = []
    for b in range(8):
        f = ((acc >> (4 * b)) & 15) if b else (acc & 15)
        new.append(counts[b] + jnp.sum(f, axis=-1, keepdims=True))
    new.append(counts[8] + jnp.sum(accq, axis=-1, keepdims=True))
    return tuple(new)


def _body(x_ref, w1_ref, b1_ref, g1_ref, be1_ref, w2_ref, b2_ref, g2_ref,
          be2_ref, o_ref):
    S = x_ref.shape[1]
    nchunks = S // C
    ngroups = nchunks // G
    tail = nchunks - ngroups * G

    init = tuple(jnp.zeros((BR, 1), jnp.int32) for _ in range(9))
    counts = jax.lax.fori_loop(
        0, ngroups, lambda g, c: _accum_group(x_ref, g * (G * C), G, c), init)
    if tail:
        counts = _accum_group(x_ref, ngroups * G * C, tail, counts)

    inv = jnp.float32(1.0 / (S_COLS + 1e-8))
    lowf = [c.astype(jnp.float32) for c in counts[:8]]
    qsum = counts[8].astype(jnp.float32)
    low_sum = lowf[0]
    low_wsum = jnp.zeros((BR, 1), jnp.float32)
    for b in range(1, 8):
        low_sum = low_sum + lowf[b]
        low_wsum = low_wsum + jnp.float32(b) * lowf[b]
    n89 = jnp.full((BR, 1), float(S_COLS), jnp.float32) - low_sum
    c9 = qsum - low_wsum - 8.0 * n89
    hist = lowf
    hist.append(n89 - c9)
    hist.append(c9)

    a1 = jnp.zeros((BR, HID), jnp.float32)
    for b in range(NBINS):
        a1 = a1 + (hist[b] * inv) * w1_ref[b:b + 1, :]
    a1 = jnp.maximum(a1 + b1_ref[...], 0.0)
    m1 = jnp.mean(a1, axis=-1, keepdims=True)
    v1 = jnp.mean((a1 - m1) ** 2, axis=-1, keepdims=True)
    h1 = (a1 - m1) * jax.lax.rsqrt(v1 + LN_EPS) * g1_ref[...] + be1_ref[...]

    a2 = jnp.dot(h1, w2_ref[...], preferred_element_type=jnp.float32)
    a2 = jnp.maximum(a2 + b2_ref[...], 0.0)
    m2 = jnp.mean(a2, axis=-1, keepdims=True)
    v2 = jnp.mean((a2 - m2) ** 2, axis=-1, keepdims=True)
    o_ref[...] = (a2 - m2) * jax.lax.rsqrt(v2 + LN_EPS) * g2_ref[...] + be2_ref[...]


def kernel(workloads, w1, b1, g1, be1, w2, b2, g2, be2):
    B, S = workloads.shape
    grid = (B // BR,)
    vec = lambda v: v.reshape(1, HID)
    out = pl.pallas_call(
        _body,
        grid=grid,
        in_specs=[
            pl.BlockSpec((BR, S), lambda i: (i, 0)),
            pl.BlockSpec((NBINS, HID), lambda i: (0, 0)),
            pl.BlockSpec((1, HID), lambda i: (0, 0)),
            pl.BlockSpec((1, HID), lambda i: (0, 0)),
            pl.BlockSpec((1, HID), lambda i: (0, 0)),
            pl.BlockSpec((HID, HID), lambda i: (0, 0)),
            pl.BlockSpec((1, HID), lambda i: (0, 0)),
            pl.BlockSpec((1, HID), lambda i: (0, 0)),
            pl.BlockSpec((1, HID), lambda i: (0, 0)),
        ],
        out_specs=pl.BlockSpec((BR, HID), lambda i: (i, 0)),
        out_shape=jax.ShapeDtypeStruct((B, HID), jnp.float32),
        compiler_params=pltpu.CompilerParams(
            dimension_semantics=("parallel",),
            vmem_limit_bytes=100 * 1024 * 1024,
        ),
    )(workloads, w1, vec(b1), vec(g1), vec(be1), w2, vec(b2), vec(g2), vec(be2))
    return out


# speedup vs baseline: 788.5198x; 1.0787x over previous
import jax
import jax.numpy as jnp
from jax.experimental import pallas as pl
from jax.experimental.pallas import tpu as pltpu

NBINS = 10
HID = 128
S_COLS = 16384
BR = 256
C = 512
G = 15
LN_EPS = 1e-5


_M4 = 0x0F0F0F0F
_M8 = 0x00FF00FF


def _accum_group(x_ref, l2e_ref, l2o_ref, g_base, nchunks, qsum):
    acc = jnp.zeros((BR, C), jnp.int32)
    accq = jnp.zeros((BR, C), jnp.int32)
    for j in range(nchunks):
        xc = x_ref[:, pl.ds(g_base + j * C, C)]
        q = xc.astype(jnp.int32)
        sh = q << 2
        p = jnp.where(xc < 8.0, jnp.int32(1) << sh, 0)
        acc = acc + p
        accq = accq + q
    l2e_ref[...] = l2e_ref[...] + (acc & _M4)
    l2o_ref[...] = l2o_ref[...] + ((acc >> 4) & _M4)
    return qsum + jnp.sum(accq, axis=-1, keepdims=True)


def _body(x_ref, w1_ref, b1_ref, g1_ref, be1_ref, w2_ref, b2_ref, g2_ref,
          be2_ref, o_ref, l2e_ref, l2o_ref):
    S = x_ref.shape[1]
    nchunks = S // C
    ngroups = nchunks // G
    tail = nchunks - ngroups * G

    l2e_ref[...] = jnp.zeros((BR, C), jnp.int32)
    l2o_ref[...] = jnp.zeros((BR, C), jnp.int32)
    qsum_i = jax.lax.fori_loop(
        0, ngroups,
        lambda g, c: _accum_group(x_ref, l2e_ref, l2o_ref, g * (G * C), G, c),
        jnp.zeros((BR, 1), jnp.int32))
    if tail:
        qsum_i = _accum_group(x_ref, l2e_ref, l2o_ref, ngroups * G * C, tail,
                              qsum_i)

    counts = [None] * 8
    us = []
    l2e = l2e_ref[...]
    l2o = l2o_ref[...]
    us.append((l2e & _M8, 0, 4))
    us.append(((l2e >> 8) & _M8, 2, 6))
    us.append((l2o & _M8, 1, 5))
    us.append(((l2o >> 8) & _M8, 3, 7))
    for u, blo, bhi in us:
        s = jnp.sum(u, axis=-1, keepdims=True)
        counts[blo] = s & 0xFFFF
        counts[bhi] = s >> 16

    inv = jnp.float32(1.0 / (S_COLS + 1e-8))
    lowf = [c.astype(jnp.float32) for c in counts[:8]]
    qsum = qsum_i.astype(jnp.float32)
    low_sum = lowf[0]
    low_wsum = jnp.zeros((BR, 1), jnp.float32)
    for b in range(1, 8):
        low_sum = low_sum + lowf[b]
        low_wsum = low_wsum + jnp.float32(b) * lowf[b]
    n89 = jnp.full((BR, 1), float(S_COLS), jnp.float32) - low_sum
    c9 = qsum - low_wsum - 8.0 * n89
    hist = lowf
    hist.append(n89 - c9)
    hist.append(c9)

    a1 = jnp.zeros((BR, HID), jnp.float32)
    for b in range(NBINS):
        a1 = a1 + (hist[b] * inv) * w1_ref[b:b + 1, :]
    a1 = jnp.maximum(a1 + b1_ref[...], 0.0)
    m1 = jnp.mean(a1, axis=-1, keepdims=True)
    v1 = jnp.mean((a1 - m1) ** 2, axis=-1, keepdims=True)
    h1 = (a1 - m1) * jax.lax.rsqrt(v1 + LN_EPS) * g1_ref[...] + be1_ref[...]

    a2 = jnp.dot(h1, w2_ref[...], preferred_element_type=jnp.float32)
    a2 = jnp.maximum(a2 + b2_ref[...], 0.0)
    m2 = jnp.mean(a2, axis=-1, keepdims=True)
    v2 = jnp.mean((a2 - m2) ** 2, axis=-1, keepdims=True)
    o_ref[...] = (a2 - m2) * jax.lax.rsqrt(v2 + LN_EPS) * g2_ref[...] + be2_ref[...]


def kernel(workloads, w1, b1, g1, be1, w2, b2, g2, be2):
    B, S = workloads.shape
    grid = (B // BR,)
    vec = lambda v: v.reshape(1, HID)
    out = pl.pallas_call(
        _body,
        grid=grid,
        in_specs=[
            pl.BlockSpec((BR, S), lambda i: (i, 0)),
            pl.BlockSpec((NBINS, HID), lambda i: (0, 0)),
            pl.BlockSpec((1, HID), lambda i: (0, 0)),
            pl.BlockSpec((1, HID), lambda i: (0, 0)),
            pl.BlockSpec((1, HID), lambda i: (0, 0)),
            pl.BlockSpec((HID, HID), lambda i: (0, 0)),
            pl.BlockSpec((1, HID), lambda i: (0, 0)),
            pl.BlockSpec((1, HID), lambda i: (0, 0)),
            pl.BlockSpec((1, HID), lambda i: (0, 0)),
        ],
        out_specs=pl.BlockSpec((BR, HID), lambda i: (i, 0)),
        out_shape=jax.ShapeDtypeStruct((B, HID), jnp.float32),
        scratch_shapes=[
            pltpu.VMEM((BR, C), jnp.int32),
            pltpu.VMEM((BR, C), jnp.int32),
        ],
        compiler_params=pltpu.CompilerParams(
            dimension_semantics=("parallel",),
            vmem_limit_bytes=100 * 1024 * 1024,
        ),
    )(workloads, w1, vec(b1), vec(g1), vec(be1), w2, vec(b2), vec(g2), vec(be2))
    return out


# speedup vs baseline: 988.1702x; 1.2532x over previous
import jax
import jax.numpy as jnp
from jax.experimental import pallas as pl
from jax.experimental.pallas import tpu as pltpu

NBINS = 10
HID = 128
S_COLS = 16384
BR = 256
C = 512
G = 15
LN_EPS = 1e-5


_M4 = 0x0F0F0F0F
_M8 = 0x00FF00FF


def _accum_group(x_ref, l2e_ref, l2o_ref, g_base, nchunks, qsum):
    acc = jnp.zeros((BR, C), jnp.int32)
    accq = jnp.zeros((BR, C), jnp.int32)
    for j in range(nchunks):
        xc = x_ref[:, pl.ds(g_base + j * C, C)]
        q = xc.astype(jnp.int32)
        sh = q << 2
        p = jnp.int32(1) << sh
        acc = acc + p
        accq = accq + q
    l2e_ref[...] = l2e_ref[...] + (acc & _M4)
    l2o_ref[...] = l2o_ref[...] + ((acc >> 4) & _M4)
    return qsum + jnp.sum(accq, axis=-1, keepdims=True)


def _body(x_ref, w1_ref, b1_ref, g1_ref, be1_ref, w2_ref, b2_ref, g2_ref,
          be2_ref, o_ref, l2e_ref, l2o_ref):
    S = x_ref.shape[1]
    nchunks = S // C
    ngroups = nchunks // G
    tail = nchunks - ngroups * G

    l2e_ref[...] = jnp.zeros((BR, C), jnp.int32)
    l2o_ref[...] = jnp.zeros((BR, C), jnp.int32)
    qsum_i = jax.lax.fori_loop(
        0, ngroups,
        lambda g, c: _accum_group(x_ref, l2e_ref, l2o_ref, g * (G * C), G, c),
        jnp.zeros((BR, 1), jnp.int32))
    if tail:
        qsum_i = _accum_group(x_ref, l2e_ref, l2o_ref, ngroups * G * C, tail,
                              qsum_i)

    counts = [None] * 8
    us = []
    l2e = l2e_ref[...]
    l2o = l2o_ref[...]
    us.append((l2e & _M8, 0, 4))
    us.append(((l2e >> 8) & _M8, 2, 6))
    us.append((l2o & _M8, 1, 5))
    us.append(((l2o >> 8) & _M8, 3, 7))
    for u, blo, bhi in us:
        s = jnp.sum(u, axis=-1, keepdims=True)
        counts[blo] = s & 0xFFFF
        counts[bhi] = s >> 16

    inv = jnp.float32(1.0 / (S_COLS + 1e-8))
    lowf = [c.astype(jnp.float32) for c in counts[:8]]
    qsum = qsum_i.astype(jnp.float32)
    low_sum = lowf[0]
    low_wsum = jnp.zeros((BR, 1), jnp.float32)
    for b in range(1, 8):
        low_sum = low_sum + lowf[b]
        low_wsum = low_wsum + jnp.float32(b) * lowf[b]
    n89 = jnp.full((BR, 1), float(S_COLS), jnp.float32) - low_sum
    c9 = qsum - low_wsum - 8.0 * n89
    hist = lowf
    hist.append(n89 - c9)
    hist.append(c9)

    a1 = jnp.zeros((BR, HID), jnp.float32)
    for b in range(NBINS):
        a1 = a1 + (hist[b] * inv) * w1_ref[b:b + 1, :]
    a1 = jnp.maximum(a1 + b1_ref[...], 0.0)
    m1 = jnp.mean(a1, axis=-1, keepdims=True)
    v1 = jnp.mean((a1 - m1) ** 2, axis=-1, keepdims=True)
    h1 = (a1 - m1) * jax.lax.rsqrt(v1 + LN_EPS) * g1_ref[...] + be1_ref[...]

    a2 = jnp.dot(h1, w2_ref[...], preferred_element_type=jnp.float32)
    a2 = jnp.maximum(a2 + b2_ref[...], 0.0)
    m2 = jnp.mean(a2, axis=-1, keepdims=True)
    v2 = jnp.mean((a2 - m2) ** 2, axis=-1, keepdims=True)
    o_ref[...] = (a2 - m2) * jax.lax.rsqrt(v2 + LN_EPS) * g2_ref[...] + be2_ref[...]


def kernel(workloads, w1, b1, g1, be1, w2, b2, g2, be2):
    B, S = workloads.shape
    grid = (B // BR,)
    vec = lambda v: v.reshape(1, HID)
    out = pl.pallas_call(
        _body,
        grid=grid,
        in_specs=[
            pl.BlockSpec((BR, S), lambda i: (i, 0)),
            pl.BlockSpec((NBINS, HID), lambda i: (0, 0)),
            pl.BlockSpec((1, HID), lambda i: (0, 0)),
            pl.BlockSpec((1, HID), lambda i: (0, 0)),
            pl.BlockSpec((1, HID), lambda i: (0, 0)),
            pl.BlockSpec((HID, HID), lambda i: (0, 0)),
            pl.BlockSpec((1, HID), lambda i: (0, 0)),
            pl.BlockSpec((1, HID), lambda i: (0, 0)),
            pl.BlockSpec((1, HID), lambda i: (0, 0)),
        ],
        out_specs=pl.BlockSpec((BR, HID), lambda i: (i, 0)),
        out_shape=jax.ShapeDtypeStruct((B, HID), jnp.float32),
        scratch_shapes=[
            pltpu.VMEM((BR, C), jnp.int32),
            pltpu.VMEM((BR, C), jnp.int32),
        ],
        compiler_params=pltpu.CompilerParams(
            dimension_semantics=("parallel",),
            vmem_limit_bytes=100 * 1024 * 1024,
        ),
    )(workloads, w1, vec(b1), vec(g1), vec(be1), w2, vec(b2), vec(g2), vec(be2))
    return out


# speedup vs baseline: 1019.6622x; 1.0319x over previous
import jax
import jax.numpy as jnp
from jax.experimental import pallas as pl
from jax.experimental.pallas import tpu as pltpu

NBINS = 10
HID = 128
S_COLS = 16384
BR = 256
C = 1024
G = 15
LN_EPS = 1e-5


_M4 = 0x0F0F0F0F
_M8 = 0x00FF00FF


def _accum_group(x_ref, l2e_ref, l2o_ref, g_base, nchunks, qsum):
    acc = jnp.zeros((BR, C), jnp.int32)
    accq = jnp.zeros((BR, C), jnp.int32)
    for j in range(nchunks):
        xc = x_ref[:, pl.ds(g_base + j * C, C)]
        q = xc.astype(jnp.int32)
        sh = q << 2
        p = jnp.int32(1) << sh
        acc = acc + p
        accq = accq + q
    l2e_ref[...] = l2e_ref[...] + (acc & _M4)
    l2o_ref[...] = l2o_ref[...] + ((acc >> 4) & _M4)
    return qsum + jnp.sum(accq, axis=-1, keepdims=True)


def _body(x_ref, w1_ref, b1_ref, g1_ref, be1_ref, w2_ref, b2_ref, g2_ref,
          be2_ref, o_ref, l2e_ref, l2o_ref):
    S = x_ref.shape[1]
    nchunks = S // C
    ngroups = nchunks // G
    tail = nchunks - ngroups * G

    l2e_ref[...] = jnp.zeros((BR, C), jnp.int32)
    l2o_ref[...] = jnp.zeros((BR, C), jnp.int32)
    qsum_i = jax.lax.fori_loop(
        0, ngroups,
        lambda g, c: _accum_group(x_ref, l2e_ref, l2o_ref, g * (G * C), G, c),
        jnp.zeros((BR, 1), jnp.int32))
    if tail:
        qsum_i = _accum_group(x_ref, l2e_ref, l2o_ref, ngroups * G * C, tail,
                              qsum_i)

    counts = [None] * 8
    us = []
    l2e = l2e_ref[...]
    l2o = l2o_ref[...]
    us.append((l2e & _M8, 0, 4))
    us.append(((l2e >> 8) & _M8, 2, 6))
    us.append((l2o & _M8, 1, 5))
    us.append(((l2o >> 8) & _M8, 3, 7))
    for u, blo, bhi in us:
        s = jnp.sum(u, axis=-1, keepdims=True)
        counts[blo] = s & 0xFFFF
        counts[bhi] = s >> 16

    inv = jnp.float32(1.0 / (S_COLS + 1e-8))
    lowf = [c.astype(jnp.float32) for c in counts[:8]]
    qsum = qsum_i.astype(jnp.float32)
    low_sum = lowf[0]
    low_wsum = jnp.zeros((BR, 1), jnp.float32)
    for b in range(1, 8):
        low_sum = low_sum + lowf[b]
        low_wsum = low_wsum + jnp.float32(b) * lowf[b]
    n89 = jnp.full((BR, 1), float(S_COLS), jnp.float32) - low_sum
    c9 = qsum - low_wsum - 8.0 * n89
    hist = lowf
    hist.append(n89 - c9)
    hist.append(c9)

    a1 = jnp.zeros((BR, HID), jnp.float32)
    for b in range(NBINS):
        a1 = a1 + (hist[b] * inv) * w1_ref[b:b + 1, :]
    a1 = jnp.maximum(a1 + b1_ref[...], 0.0)
    m1 = jnp.mean(a1, axis=-1, keepdims=True)
    v1 = jnp.mean((a1 - m1) ** 2, axis=-1, keepdims=True)
    h1 = (a1 - m1) * jax.lax.rsqrt(v1 + LN_EPS) * g1_ref[...] + be1_ref[...]

    a2 = jnp.dot(h1, w2_ref[...], preferred_element_type=jnp.float32)
    a2 = jnp.maximum(a2 + b2_ref[...], 0.0)
    m2 = jnp.mean(a2, axis=-1, keepdims=True)
    v2 = jnp.mean((a2 - m2) ** 2, axis=-1, keepdims=True)
    o_ref[...] = (a2 - m2) * jax.lax.rsqrt(v2 + LN_EPS) * g2_ref[...] + be2_ref[...]


def kernel(workloads, w1, b1, g1, be1, w2, b2, g2, be2):
    B, S = workloads.shape
    grid = (B // BR,)
    vec = lambda v: v.reshape(1, HID)
    out = pl.pallas_call(
        _body,
        grid=grid,
        in_specs=[
            pl.BlockSpec((BR, S), lambda i: (i, 0)),
            pl.BlockSpec((NBINS, HID), lambda i: (0, 0)),
            pl.BlockSpec((1, HID), lambda i: (0, 0)),
            pl.BlockSpec((1, HID), lambda i: (0, 0)),
            pl.BlockSpec((1, HID), lambda i: (0, 0)),
            pl.BlockSpec((HID, HID), lambda i: (0, 0)),
            pl.BlockSpec((1, HID), lambda i: (0, 0)),
            pl.BlockSpec((1, HID), lambda i: (0, 0)),
            pl.BlockSpec((1, HID), lambda i: (0, 0)),
        ],
        out_specs=pl.BlockSpec((BR, HID), lambda i: (i, 0)),
        out_shape=jax.ShapeDtypeStruct((B, HID), jnp.float32),
        scratch_shapes=[
            pltpu.VMEM((BR, C), jnp.int32),
            pltpu.VMEM((BR, C), jnp.int32),
        ],
        compiler_params=pltpu.CompilerParams(
            dimension_semantics=("parallel",),
            vmem_limit_bytes=100 * 1024 * 1024,
        ),
    )(workloads, w1, vec(b1), vec(g1), vec(be1), w2, vec(b2), vec(g2), vec(be2))
    return out
